# baseline (speedup 1.0000x reference)
import jax
import jax.numpy as jnp
from jax import lax
from jax.experimental import pallas as pl
from jax.experimental.pallas import tpu as pltpu
from jax.experimental.pallas import tpu_sc as plsc

N = 10000
E = 320000
IN_DIM = 128
H0, D0 = 8, 16
SLOPE = 0.2

NC, NS = 2, 16
NW = NC * NS
EPW = E // NW
SLAB = 624
REM = N - NS * SLAB

B0, CH0 = 40, 25
NBLK0 = EPW // B0
B1, CH1 = 80, 25
NBLK1 = EPW // B1

BM = 400
GRID = N // BM

F0 = 144
F1 = 48
F1C = 64


def _leaky(x):
    return jnp.maximum(x, x * SLOPE)


def _splat16(v, h):
    idx = jnp.full((16, 1), h, dtype=jnp.int32)
    return lax.gather(
        v, idx,
        lax.GatherDimensionNumbers(
            offset_dims=(), collapsed_slice_dims=(0,), start_index_map=(0,)),
        slice_sizes=(1,),
        mode=lax.GatherScatterMode.PROMISE_IN_BOUNDS)


def _zero_slab(num_sh, msg0, s, width):
    nb = msg0.shape[0]
    zero16 = jnp.zeros((16,), jnp.float32)

    def _zrow(i, carry):
        for j in range(width // 16):
            msg0[i, pl.ds(j * 16, 16)] = zero16
        return carry
    lax.fori_loop(0, nb, _zrow, 0)

    r0 = s * SLAB
    for k in range(SLAB // nb):
        pltpu.sync_copy(msg0, num_sh.at[pl.ds(r0 + k * nb, nb)])
    if SLAB % nb:
        pltpu.sync_copy(msg0.at[pl.ds(0, SLAB % nb)],
                        num_sh.at[pl.ds(r0 + (SLAB // nb) * nb, SLAB % nb)])

    @pl.when(s == NS - 1)
    def _zrem():
        pltpu.sync_copy(msg0.at[pl.ds(0, REM)],
                        num_sh.at[pl.ds(NS * SLAB, REM)])


def _writeback(num_sh, out_hbm, c, s):
    r0 = s * SLAB
    pltpu.sync_copy(num_sh.at[pl.ds(r0, SLAB)], out_hbm.at[c, pl.ds(r0, SLAB)])

    @pl.when(s == NS - 1)
    def _wrem():
        pltpu.sync_copy(num_sh.at[pl.ds(NS * SLAB, REM)],
                        out_hbm.at[c, pl.ds(NS * SLAB, REM)])


def _mm0_body(x_ref, w_ref, al_ref, ar_ref, comb_ref, ert_ref):
    feat = jnp.dot(x_ref[...], w_ref[...], preferred_element_type=jnp.float32)
    el = jnp.dot(feat, al_ref[...], preferred_element_type=jnp.float32)
    er = jnp.dot(feat, ar_ref[...], preferred_element_type=jnp.float32)
    comb_ref[:, :128] = feat
    comb_ref[:, 128:136] = el
    comb_ref[:, 136:144] = er
    ert_ref[:, :8] = er
    ert_ref[:, 8:] = jnp.zeros((BM, 8), jnp.float32)


def _mm0(x, w0, al, ar):
    return pl.pallas_call(
        _mm0_body,
        grid=(GRID,),
        in_specs=[
            pl.BlockSpec((BM, IN_DIM), lambda i: (i, 0)),
            pl.BlockSpec((IN_DIM, IN_DIM), lambda i: (0, 0)),
            pl.BlockSpec((IN_DIM, 8), lambda i: (0, 0)),
            pl.BlockSpec((IN_DIM, 8), lambda i: (0, 0)),
        ],
        out_specs=[
            pl.BlockSpec((BM, F0), lambda i: (i, 0)),
            pl.BlockSpec((BM, 16), lambda i: (i, 0)),
        ],
        out_shape=[
            jax.ShapeDtypeStruct((N, F0), jnp.float32),
            jax.ShapeDtypeStruct((N, 16), jnp.float32),
        ],
    )(x, w0, al, ar)


def _edge0_body(comb_hbm, ert_hbm, src_hbm, dst_hbm, out_hbm,
                num_sh, sidx, didx, comb_b, er_b, msg_b,
                sg0, sg1, sg2, ss0, ss1, ss2):
    c = lax.axis_index("c")
    s = lax.axis_index("s")
    wid = c * NS + s
    semg = (sg0, sg1, sg2)
    sems = (ss0, ss1, ss2)

    _zero_slab(num_sh, msg_b.at[0], s, F0)
    plsc.subcore_barrier()

    def _row(buf, b):
        return buf.at[(b // CH0) % 2, b % CH0]

    def _refill(cnext):
        par = cnext % 2
        pltpu.sync_copy(src_hbm.at[wid, pl.ds(cnext * CH0, CH0)],
                        sidx.at[par])
        pltpu.sync_copy(dst_hbm.at[wid, pl.ds(cnext * CH0, CH0)],
                        didx.at[par])

    mask8 = lax.iota(jnp.int32, 16) < 8

    def g_issue(j, p):
        pltpu.async_copy(comb_hbm.at[_row(sidx, j)], comb_b.at[p], semg[p])
        pltpu.async_copy(ert_hbm.at[_row(didx, j)], er_b.at[p], semg[p])

    def g_wait(j, p):
        pltpu.make_async_copy(comb_hbm.at[_row(sidx, j)], comb_b.at[p],
                              semg[p]).wait()
        pltpu.make_async_copy(ert_hbm.at[_row(didx, j)], er_b.at[p],
                              semg[p]).wait()

    def s_issue(j, p):
        pltpu.async_copy(msg_b.at[p], num_sh.at[_row(didx, j)], sems[p],
                         add=True)

    def s_wait(j, p):
        pltpu.make_async_copy(msg_b.at[p], num_sh.at[_row(didx, j)],
                              sems[p]).wait()

    def compute(p):
        def _edge(i):
            tail = comb_b[p, i, pl.ds(128, 16)]
            erd = er_b[p, i, :]
            w = jnp.exp(_leaky(tail + erd))
            w = jnp.where(mask8, w, 0.0)
            msg_b[p, i, pl.ds(128, 16)] = w
            for h in range(H0):
                wh = _splat16(w, h)
                msg_b[p, i, pl.ds(h * 16, 16)] = (
                    comb_b[p, i, pl.ds(h * 16, 16)] * wh)
        plsc.parallel_loop(0, B0, 1, unroll=16)(_edge)

    D = 3
    _refill(0)
    for k in range(D - 1):
        g_issue(k, k)

    def _outer(jo, carry):
        for u in range(D):
            j = D * jo + u
            if u == 1:
                jm = j % CH0
                @pl.when((jm >= CH0 // 2) & (jm < CH0 // 2 + D)
                         & (j < (NBLK0 // CH0 - 1) * CH0))
                def _rf():
                    _refill(j // CH0 + 1)

            @pl.when(j + D - 1 < NBLK0)
            def _gi():
                g_issue(j + D - 1, (u + D - 1) % D)
            g_wait(j, u)

            @pl.when(jo >= 1)
            def _ws():
                s_wait(j - D, u)
            compute(u)
            s_issue(j, u)
        return carry
    lax.fori_loop(0, NBLK0 // D, _outer, 0)

    for t in range(NBLK0 % D):
        j = (NBLK0 // D) * D + t
        g_wait(j, j % D)
        s_wait(j - D, j % D)
        compute(j % D)
        s_issue(j, j % D)
    for t in range(D):
        s_wait(NBLK0 - D + t, (NBLK0 - D + t) % D)

    plsc.subcore_barrier()
    _writeback(num_sh, out_hbm, c, s)


def _edge0(comb0, ert0, src, dst):
    mesh = plsc.VectorSubcoreMesh(core_axis_name="c", subcore_axis_name="s",
                                  num_cores=NC, num_subcores=NS)
    f = pl.kernel(
        _edge0_body,
        out_type=jax.ShapeDtypeStruct((NC, N, F0), jnp.float32),
        mesh=mesh,
        scratch_types=[
            pltpu.VMEM_SHARED((N, F0), jnp.float32),
            pltpu.VMEM((2, CH0, B0), jnp.int32),
            pltpu.VMEM((2, CH0, B0), jnp.int32),
            pltpu.VMEM((3, B0, F0), jnp.float32),
            pltpu.VMEM((3, B0, 16), jnp.float32),
            pltpu.VMEM((3, B0, F0), jnp.float32),
            pltpu.SemaphoreType.DMA,
            pltpu.SemaphoreType.DMA,
            pltpu.SemaphoreType.DMA,
            pltpu.SemaphoreType.DMA,
            pltpu.SemaphoreType.DMA,
            pltpu.SemaphoreType.DMA,
        ],
        compiler_params=pltpu.CompilerParams(use_tc_tiling_on_sc=False),
    )
    return f(comb0, ert0, src, dst)


def _mm1_body(np_ref, w1_ref, alm_ref, arm_ref, exp8_ref,
              h_ref, comb1_ref, er_ref):
    a = np_ref[0] + np_ref[1]
    num = a[:, :128]
    den8 = a[:, 128:136]
    den = jnp.dot(den8, exp8_ref[...], preferred_element_type=jnp.float32)
    hv = num / (den + 1e-9)
    hv = jnp.where(hv > 0, hv, jnp.exp(jnp.minimum(hv, 0.0)) - 1.0)
    h_ref[...] = hv
    feat1 = jnp.dot(hv, w1_ref[...], preferred_element_type=jnp.float32)
    comb1_ref[:, :40] = feat1
    comb1_ref[:, 40:41] = jnp.ones((BM, 1), jnp.float32)
    comb1_ref[:, 41:48] = jnp.zeros((BM, 7), jnp.float32)
    comb1_ref[:, 48:64] = jnp.dot(feat1, alm_ref[...],
                                  preferred_element_type=jnp.float32)
    er_ref[...] = jnp.dot(feat1, arm_ref[...], preferred_element_type=jnp.float32)


def _mm1(np0, w1, alm, arm, exp8):
    return pl.pallas_call(
        _mm1_body,
        grid=(GRID,),
        in_specs=[
            pl.BlockSpec((NC, BM, F0), lambda i: (0, i, 0)),
            pl.BlockSpec((IN_DIM, 40), lambda i: (0, 0)),
            pl.BlockSpec((40, 16), lambda i: (0, 0)),
            pl.BlockSpec((40, 16), lambda i: (0, 0)),
            pl.BlockSpec((8, 128), lambda i: (0, 0)),
        ],
        out_specs=[
            pl.BlockSpec((BM, IN_DIM), lambda i: (i, 0)),
            pl.BlockSpec((BM, F1C), lambda i: (i, 0)),
            pl.BlockSpec((BM, 16), lambda i: (i, 0)),
        ],
        out_shape=[
            jax.ShapeDtypeStruct((N, IN_DIM), jnp.float32),
            jax.ShapeDtypeStruct((N, F1C), jnp.float32),
            jax.ShapeDtypeStruct((N, 16), jnp.float32),
        ],
    )(np0, w1, alm, arm, exp8)


def _edge1_body(comb_hbm, er_hbm, src_hbm, dst_hbm, out_hbm,
                num_sh, sidx, didx, comb_b, er_b, msg_b,
                sg0, sg1, sg2, sg3, ss0, ss1, ss2, ss3):
    c = lax.axis_index("c")
    s = lax.axis_index("s")
    wid = c * NS + s
    semg = (sg0, sg1, sg2, sg3)
    sems = (ss0, ss1, ss2, ss3)

    _zero_slab(num_sh, msg_b.at[0], s, F1)
    plsc.subcore_barrier()

    def _row(buf, b):
        return buf.at[(b // CH1) % 2, b % CH1]

    def _refill(cnext):
        par = cnext % 2
        pltpu.sync_copy(src_hbm.at[wid, pl.ds(cnext * CH1, CH1)],
                        sidx.at[par])
        pltpu.sync_copy(dst_hbm.at[wid, pl.ds(cnext * CH1, CH1)],
                        didx.at[par])

    def g_issue(j, p):
        pltpu.async_copy(comb_hbm.at[_row(sidx, j)], comb_b.at[p], semg[p])
        pltpu.async_copy(er_hbm.at[_row(didx, j)], er_b.at[p], semg[p])

    def g_wait(j, p):
        pltpu.make_async_copy(comb_hbm.at[_row(sidx, j)], comb_b.at[p],
                              semg[p]).wait()
        pltpu.make_async_copy(er_hbm.at[_row(didx, j)], er_b.at[p],
                              semg[p]).wait()

    def s_issue(j, p):
        pltpu.async_copy(msg_b.at[p], num_sh.at[_row(didx, j)], sems[p],
                         add=True)

    def s_wait(j, p):
        pltpu.make_async_copy(msg_b.at[p], num_sh.at[_row(didx, j)],
                              sems[p]).wait()

    def compute(p):
        def _edge(i):
            w = jnp.exp(_leaky(comb_b[p, i, pl.ds(48, 16)] + er_b[p, i, :]))
            for t in range(F1 // 16):
                msg_b[p, i, pl.ds(t * 16, 16)] = (
                    comb_b[p, i, pl.ds(t * 16, 16)] * w)
        plsc.parallel_loop(0, B1, 1, unroll=16)(_edge)

    D = 4
    _refill(0)
    for k in range(D - 1):
        g_issue(k, k)

    def _outer(jo, carry):
        for u in range(D):
            j = D * jo + u
            if u == 1:
                jm = j % CH1
                @pl.when((jm >= CH1 // 2) & (jm < CH1 // 2 + D)
                         & (j < (NBLK1 // CH1 - 1) * CH1))
                def _rf():
                    _refill(j // CH1 + 1)

            @pl.when(j + D - 1 < NBLK1)
            def _gi():
                g_issue(j + D - 1, (u + D - 1) % D)
            g_wait(j, u)

            @pl.when(jo >= 1)
            def _ws():
                s_wait(j - D, u)
            compute(u)
            s_issue(j, u)
        return carry
    lax.fori_loop(0, NBLK1 // D, _outer, 0)

    for t in range(NBLK1 % D):
        j = (NBLK1 // D) * D + t
        g_wait(j, j % D)
        s_wait(j - D, j % D)
        compute(j % D)
        s_issue(j, j % D)
    for t in range(D):
        s_wait(NBLK1 - D + t, (NBLK1 - D + t) % D)

    plsc.subcore_barrier()
    _writeback(num_sh, out_hbm, c, s)


def _edge1(comb1, er1, src, dst):
    mesh = plsc.VectorSubcoreMesh(core_axis_name="c", subcore_axis_name="s",
                                  num_cores=NC, num_subcores=NS)
    f = pl.kernel(
        _edge1_body,
        out_type=jax.ShapeDtypeStruct((NC, N, F1), jnp.float32),
        mesh=mesh,
        scratch_types=[
            pltpu.VMEM_SHARED((N, F1), jnp.float32),
            pltpu.VMEM((2, CH1, B1), jnp.int32),
            pltpu.VMEM((2, CH1, B1), jnp.int32),
            pltpu.VMEM((4, B1, F1C), jnp.float32),
            pltpu.VMEM((4, B1, 16), jnp.float32),
            pltpu.VMEM((4, B1, F1), jnp.float32),
            pltpu.SemaphoreType.DMA,
            pltpu.SemaphoreType.DMA,
            pltpu.SemaphoreType.DMA,
            pltpu.SemaphoreType.DMA,
            pltpu.SemaphoreType.DMA,
            pltpu.SemaphoreType.DMA,
            pltpu.SemaphoreType.DMA,
            pltpu.SemaphoreType.DMA,
        ],
        compiler_params=pltpu.CompilerParams(use_tc_tiling_on_sc=False),
    )
    return f(comb1, er1, src, dst)


def _fin_body(np_ref, out_ref):
    a = np_ref[0] + np_ref[1]
    den = a[:, 40:41]
    logits = a[:, :40] / (den + 1e-9)
    m = jnp.max(logits, axis=1, keepdims=True)
    sh = logits - m
    lse = jnp.log(jnp.sum(jnp.exp(sh), axis=1, keepdims=True))
    out_ref[...] = sh - lse


def _fin(np1):
    return pl.pallas_call(
        _fin_body,
        grid=(GRID,),
        in_specs=[pl.BlockSpec((NC, BM, F1), lambda i: (0, i, 0))],
        out_specs=pl.BlockSpec((BM, 40), lambda i: (i, 0)),
        out_shape=jax.ShapeDtypeStruct((N, 40), jnp.float32),
    )(np1)


def kernel(inputs, edge_index, W0, al0, ar0, W1, al1, ar1):
    src0 = edge_index[0].reshape(NW, NBLK0, B0)
    dst0 = edge_index[1].reshape(NW, NBLK0, B0)
    src1 = edge_index[0].reshape(NW, NBLK1, B1)
    dst1 = edge_index[1].reshape(NW, NBLK1, B1)

    eye8 = jnp.eye(H0, dtype=jnp.float32)
    Al = (al0[:, :, None] * eye8[:, None, :]).reshape(H0 * D0, H0)
    Ar = (ar0[:, :, None] * eye8[:, None, :]).reshape(H0 * D0, H0)
    alm = jnp.broadcast_to(al1[0][:, None], (40, 16))
    arm = jnp.broadcast_to(ar1[0][:, None], (40, 16))
    exp8 = jnp.repeat(eye8, D0, axis=1)

    comb0, ert0 = _mm0(inputs, W0, Al, Ar)
    np0 = _edge0(comb0, ert0, src0, dst0)
    h, comb1, er1 = _mm1(np0, W1, alm, arm, exp8)
    np1 = _edge1(comb1, er1, src1, dst1)
    out = _fin(np1)
    return (out, h)


# speedup vs baseline: 1.0146x; 1.0146x over previous
import jax
import jax.numpy as jnp
from jax import lax
from jax.experimental import pallas as pl
from jax.experimental.pallas import tpu as pltpu
from jax.experimental.pallas import tpu_sc as plsc

N = 10000
E = 320000
IN_DIM = 128
H0, D0 = 8, 16
SLOPE = 0.2

NC, NS = 2, 16
NW = NC * NS
EPW = E // NW
SLAB = 624
REM = N - NS * SLAB

B0, CH0 = 40, 25
NBLK0 = EPW // B0
B1, CH1 = 80, 25
NBLK1 = EPW // B1

BM = 400
GRID = N // BM

F0 = 144
F1 = 48
F1C = 64


def _leaky(x):
    return jnp.maximum(x, x * SLOPE)


def _splat16(v, h):
    idx = jnp.full((16, 1), h, dtype=jnp.int32)
    return lax.gather(
        v, idx,
        lax.GatherDimensionNumbers(
            offset_dims=(), collapsed_slice_dims=(0,), start_index_map=(0,)),
        slice_sizes=(1,),
        mode=lax.GatherScatterMode.PROMISE_IN_BOUNDS)


def _zero_slab(num_sh, msg0, s, width):
    nb = msg0.shape[0]
    zero16 = jnp.zeros((16,), jnp.float32)

    def _zrow(i, carry):
        for j in range(width // 16):
            msg0[i, pl.ds(j * 16, 16)] = zero16
        return carry
    lax.fori_loop(0, nb, _zrow, 0)

    r0 = s * SLAB
    for k in range(SLAB // nb):
        pltpu.sync_copy(msg0, num_sh.at[pl.ds(r0 + k * nb, nb)])
    if SLAB % nb:
        pltpu.sync_copy(msg0.at[pl.ds(0, SLAB % nb)],
                        num_sh.at[pl.ds(r0 + (SLAB // nb) * nb, SLAB % nb)])

    @pl.when(s == NS - 1)
    def _zrem():
        pltpu.sync_copy(msg0.at[pl.ds(0, REM)],
                        num_sh.at[pl.ds(NS * SLAB, REM)])


def _writeback(num_sh, out_hbm, c, s):
    r0 = s * SLAB
    pltpu.sync_copy(num_sh.at[pl.ds(r0, SLAB)], out_hbm.at[c, pl.ds(r0, SLAB)])

    @pl.when(s == NS - 1)
    def _wrem():
        pltpu.sync_copy(num_sh.at[pl.ds(NS * SLAB, REM)],
                        out_hbm.at[c, pl.ds(NS * SLAB, REM)])


def _mm0_body(x_ref, w_ref, al_ref, ar_ref, comb_ref, ert_ref):
    feat = jnp.dot(x_ref[...], w_ref[...], preferred_element_type=jnp.float32)
    el = jnp.dot(feat, al_ref[...], preferred_element_type=jnp.float32)
    er = jnp.dot(feat, ar_ref[...], preferred_element_type=jnp.float32)
    comb_ref[:, :128] = feat
    comb_ref[:, 128:136] = el
    comb_ref[:, 136:144] = er
    ert_ref[:, :8] = er
    ert_ref[:, 8:] = jnp.zeros((BM, 8), jnp.float32)


def _mm0(x, w0, al, ar):
    return pl.pallas_call(
        _mm0_body,
        grid=(GRID,),
        in_specs=[
            pl.BlockSpec((BM, IN_DIM), lambda i: (i, 0)),
            pl.BlockSpec((IN_DIM, IN_DIM), lambda i: (0, 0)),
            pl.BlockSpec((IN_DIM, 8), lambda i: (0, 0)),
            pl.BlockSpec((IN_DIM, 8), lambda i: (0, 0)),
        ],
        out_specs=[
            pl.BlockSpec((BM, F0), lambda i: (i, 0)),
            pl.BlockSpec((BM, 16), lambda i: (i, 0)),
        ],
        out_shape=[
            jax.ShapeDtypeStruct((N, F0), jnp.float32),
            jax.ShapeDtypeStruct((N, 16), jnp.float32),
        ],
    )(x, w0, al, ar)


def _edge0_body(comb_hbm, ert_hbm, src_hbm, dst_hbm, out_hbm,
                num_sh, sidx, didx, comb_b, er_b, msg_b,
                sg0, sg1, sg2, ss0, ss1, ss2):
    c = lax.axis_index("c")
    s = lax.axis_index("s")
    wid = c * NS + s
    semg = (sg0, sg1, sg2)
    sems = (ss0, ss1, ss2)

    _zero_slab(num_sh, msg_b.at[0], s, F0)
    plsc.subcore_barrier()

    def _row(buf, b):
        return buf.at[(b // CH0) % 2, b % CH0]

    def _refill(cnext):
        par = cnext % 2
        pltpu.sync_copy(src_hbm.at[wid, pl.ds(cnext * CH0, CH0)],
                        sidx.at[par])
        pltpu.sync_copy(dst_hbm.at[wid, pl.ds(cnext * CH0, CH0)],
                        didx.at[par])

    mask8 = lax.iota(jnp.int32, 16) < 8

    def g_issue(j, p):
        pltpu.async_copy(comb_hbm.at[_row(sidx, j)], comb_b.at[p], semg[p])
        pltpu.async_copy(ert_hbm.at[_row(didx, j)], er_b.at[p], semg[p])

    def g_wait(j, p):
        pltpu.make_async_copy(comb_hbm.at[_row(sidx, j)], comb_b.at[p],
                              semg[p]).wait()
        pltpu.make_async_copy(ert_hbm.at[_row(didx, j)], er_b.at[p],
                              semg[p]).wait()

    def s_issue(j, p):
        pltpu.async_copy(msg_b.at[p], num_sh.at[_row(didx, j)], sems[p],
                         add=True)

    def s_wait(j, p):
        pltpu.make_async_copy(msg_b.at[p], num_sh.at[_row(didx, j)],
                              sems[p]).wait()

    def compute(p):
        def _edge(i):
            tail = comb_b[p, i, pl.ds(128, 16)]
            erd = er_b[p, i, :]
            w = jnp.exp(_leaky(tail + erd))
            w = jnp.where(mask8, w, 0.0)
            msg_b[p, i, pl.ds(128, 16)] = w
            for h in range(H0):
                wh = _splat16(w, h)
                msg_b[p, i, pl.ds(h * 16, 16)] = (
                    comb_b[p, i, pl.ds(h * 16, 16)] * wh)
        plsc.parallel_loop(0, B0, 1, unroll=8)(_edge)

    D = 3
    _refill(0)
    for k in range(D - 1):
        g_issue(k, k)

    def _outer(jo, carry):
        for u in range(D):
            j = D * jo + u
            if u == 1:
                jm = j % CH0
                @pl.when((jm >= CH0 // 2) & (jm < CH0 // 2 + D)
                         & (j < (NBLK0 // CH0 - 1) * CH0))
                def _rf():
                    _refill(j // CH0 + 1)

            @pl.when(j + D - 1 < NBLK0)
            def _gi():
                g_issue(j + D - 1, (u + D - 1) % D)
            g_wait(j, u)

            @pl.when(jo >= 1)
            def _ws():
                s_wait(j - D, u)
            compute(u)
            s_issue(j, u)
        return carry
    lax.fori_loop(0, NBLK0 // D, _outer, 0)

    for t in range(NBLK0 % D):
        j = (NBLK0 // D) * D + t
        g_wait(j, j % D)
        s_wait(j - D, j % D)
        compute(j % D)
        s_issue(j, j % D)
    for t in range(D):
        s_wait(NBLK0 - D + t, (NBLK0 - D + t) % D)

    plsc.subcore_barrier()
    _writeback(num_sh, out_hbm, c, s)


def _edge0(comb0, ert0, src, dst):
    mesh = plsc.VectorSubcoreMesh(core_axis_name="c", subcore_axis_name="s",
                                  num_cores=NC, num_subcores=NS)
    f = pl.kernel(
        _edge0_body,
        out_type=jax.ShapeDtypeStruct((NC, N, F0), jnp.float32),
        mesh=mesh,
        scratch_types=[
            pltpu.VMEM_SHARED((N, F0), jnp.float32),
            pltpu.VMEM((2, CH0, B0), jnp.int32),
            pltpu.VMEM((2, CH0, B0), jnp.int32),
            pltpu.VMEM((3, B0, F0), jnp.float32),
            pltpu.VMEM((3, B0, 16), jnp.float32),
            pltpu.VMEM((3, B0, F0), jnp.float32),
            pltpu.SemaphoreType.DMA,
            pltpu.SemaphoreType.DMA,
            pltpu.SemaphoreType.DMA,
            pltpu.SemaphoreType.DMA,
            pltpu.SemaphoreType.DMA,
            pltpu.SemaphoreType.DMA,
        ],
        compiler_params=pltpu.CompilerParams(use_tc_tiling_on_sc=False),
    )
    return f(comb0, ert0, src, dst)


def _mm1_body(np_ref, w1_ref, alm_ref, arm_ref, exp8_ref,
              h_ref, comb1_ref, er_ref):
    a = np_ref[0] + np_ref[1]
    num = a[:, :128]
    den8 = a[:, 128:136]
    den = jnp.dot(den8, exp8_ref[...], preferred_element_type=jnp.float32)
    hv = num / (den + 1e-9)
    hv = jnp.where(hv > 0, hv, jnp.exp(jnp.minimum(hv, 0.0)) - 1.0)
    h_ref[...] = hv
    feat1 = jnp.dot(hv, w1_ref[...], preferred_element_type=jnp.float32)
    comb1_ref[:, :40] = feat1
    comb1_ref[:, 40:41] = jnp.ones((BM, 1), jnp.float32)
    comb1_ref[:, 41:48] = jnp.zeros((BM, 7), jnp.float32)
    comb1_ref[:, 48:64] = jnp.dot(feat1, alm_ref[...],
                                  preferred_element_type=jnp.float32)
    er_ref[...] = jnp.dot(feat1, arm_ref[...], preferred_element_type=jnp.float32)


def _mm1(np0, w1, alm, arm, exp8):
    return pl.pallas_call(
        _mm1_body,
        grid=(GRID,),
        in_specs=[
            pl.BlockSpec((NC, BM, F0), lambda i: (0, i, 0)),
            pl.BlockSpec((IN_DIM, 40), lambda i: (0, 0)),
            pl.BlockSpec((40, 16), lambda i: (0, 0)),
            pl.BlockSpec((40, 16), lambda i: (0, 0)),
            pl.BlockSpec((8, 128), lambda i: (0, 0)),
        ],
        out_specs=[
            pl.BlockSpec((BM, IN_DIM), lambda i: (i, 0)),
            pl.BlockSpec((BM, F1C), lambda i: (i, 0)),
            pl.BlockSpec((BM, 16), lambda i: (i, 0)),
        ],
        out_shape=[
            jax.ShapeDtypeStruct((N, IN_DIM), jnp.float32),
            jax.ShapeDtypeStruct((N, F1C), jnp.float32),
            jax.ShapeDtypeStruct((N, 16), jnp.float32),
        ],
    )(np0, w1, alm, arm, exp8)


def _edge1_body(comb_hbm, er_hbm, src_hbm, dst_hbm, out_hbm,
                num_sh, sidx, didx, comb_b, er_b, msg_b,
                sg0, sg1, sg2, sg3, ss0, ss1, ss2, ss3):
    c = lax.axis_index("c")
    s = lax.axis_index("s")
    wid = c * NS + s
    semg = (sg0, sg1, sg2, sg3)
    sems = (ss0, ss1, ss2, ss3)

    _zero_slab(num_sh, msg_b.at[0], s, F1)
    plsc.subcore_barrier()

    def _row(buf, b):
        return buf.at[(b // CH1) % 2, b % CH1]

    def _refill(cnext):
        par = cnext % 2
        pltpu.sync_copy(src_hbm.at[wid, pl.ds(cnext * CH1, CH1)],
                        sidx.at[par])
        pltpu.sync_copy(dst_hbm.at[wid, pl.ds(cnext * CH1, CH1)],
                        didx.at[par])

    def g_issue(j, p):
        pltpu.async_copy(comb_hbm.at[_row(sidx, j)], comb_b.at[p], semg[p])
        pltpu.async_copy(er_hbm.at[_row(didx, j)], er_b.at[p], semg[p])

    def g_wait(j, p):
        pltpu.make_async_copy(comb_hbm.at[_row(sidx, j)], comb_b.at[p],
                              semg[p]).wait()
        pltpu.make_async_copy(er_hbm.at[_row(didx, j)], er_b.at[p],
                              semg[p]).wait()

    def s_issue(j, p):
        pltpu.async_copy(msg_b.at[p], num_sh.at[_row(didx, j)], sems[p],
                         add=True)

    def s_wait(j, p):
        pltpu.make_async_copy(msg_b.at[p], num_sh.at[_row(didx, j)],
                              sems[p]).wait()

    def compute(p):
        def _edge(i):
            w = jnp.exp(_leaky(comb_b[p, i, pl.ds(48, 16)] + er_b[p, i, :]))
            for t in range(F1 // 16):
                msg_b[p, i, pl.ds(t * 16, 16)] = (
                    comb_b[p, i, pl.ds(t * 16, 16)] * w)
        plsc.parallel_loop(0, B1, 1, unroll=8)(_edge)

    D = 4
    _refill(0)
    for k in range(D - 1):
        g_issue(k, k)

    def _outer(jo, carry):
        for u in range(D):
            j = D * jo + u
            if u == 1:
                jm = j % CH1
                @pl.when((jm >= CH1 // 2) & (jm < CH1 // 2 + D)
                         & (j < (NBLK1 // CH1 - 1) * CH1))
                def _rf():
                    _refill(j // CH1 + 1)

            @pl.when(j + D - 1 < NBLK1)
            def _gi():
                g_issue(j + D - 1, (u + D - 1) % D)
            g_wait(j, u)

            @pl.when(jo >= 1)
            def _ws():
                s_wait(j - D, u)
            compute(u)
            s_issue(j, u)
        return carry
    lax.fori_loop(0, NBLK1 // D, _outer, 0)

    for t in range(NBLK1 % D):
        j = (NBLK1 // D) * D + t
        g_wait(j, j % D)
        s_wait(j - D, j % D)
        compute(j % D)
        s_issue(j, j % D)
    for t in range(D):
        s_wait(NBLK1 - D + t, (NBLK1 - D + t) % D)

    plsc.subcore_barrier()
    _writeback(num_sh, out_hbm, c, s)


def _edge1(comb1, er1, src, dst):
    mesh = plsc.VectorSubcoreMesh(core_axis_name="c", subcore_axis_name="s",
                                  num_cores=NC, num_subcores=NS)
    f = pl.kernel(
        _edge1_body,
        out_type=jax.ShapeDtypeStruct((NC, N, F1), jnp.float32),
        mesh=mesh,
        scratch_types=[
            pltpu.VMEM_SHARED((N, F1), jnp.float32),
            pltpu.VMEM((2, CH1, B1), jnp.int32),
            pltpu.VMEM((2, CH1, B1), jnp.int32),
            pltpu.VMEM((4, B1, F1C), jnp.float32),
            pltpu.VMEM((4, B1, 16), jnp.float32),
            pltpu.VMEM((4, B1, F1), jnp.float32),
            pltpu.SemaphoreType.DMA,
            pltpu.SemaphoreType.DMA,
            pltpu.SemaphoreType.DMA,
            pltpu.SemaphoreType.DMA,
            pltpu.SemaphoreType.DMA,
            pltpu.SemaphoreType.DMA,
            pltpu.SemaphoreType.DMA,
            pltpu.SemaphoreType.DMA,
        ],
        compiler_params=pltpu.CompilerParams(use_tc_tiling_on_sc=False),
    )
    return f(comb1, er1, src, dst)


def _fin_body(np_ref, out_ref):
    a = np_ref[0] + np_ref[1]
    den = a[:, 40:41]
    logits = a[:, :40] / (den + 1e-9)
    m = jnp.max(logits, axis=1, keepdims=True)
    sh = logits - m
    lse = jnp.log(jnp.sum(jnp.exp(sh), axis=1, keepdims=True))
    out_ref[...] = sh - lse


def _fin(np1):
    return pl.pallas_call(
        _fin_body,
        grid=(GRID,),
        in_specs=[pl.BlockSpec((NC, BM, F1), lambda i: (0, i, 0))],
        out_specs=pl.BlockSpec((BM, 40), lambda i: (i, 0)),
        out_shape=jax.ShapeDtypeStruct((N, 40), jnp.float32),
    )(np1)


def kernel(inputs, edge_index, W0, al0, ar0, W1, al1, ar1):
    src0 = edge_index[0].reshape(NW, NBLK0, B0)
    dst0 = edge_index[1].reshape(NW, NBLK0, B0)
    src1 = edge_index[0].reshape(NW, NBLK1, B1)
    dst1 = edge_index[1].reshape(NW, NBLK1, B1)

    eye8 = jnp.eye(H0, dtype=jnp.float32)
    Al = (al0[:, :, None] * eye8[:, None, :]).reshape(H0 * D0, H0)
    Ar = (ar0[:, :, None] * eye8[:, None, :]).reshape(H0 * D0, H0)
    alm = jnp.broadcast_to(al1[0][:, None], (40, 16))
    arm = jnp.broadcast_to(ar1[0][:, None], (40, 16))
    exp8 = jnp.repeat(eye8, D0, axis=1)

    comb0, ert0 = _mm0(inputs, W0, Al, Ar)
    np0 = _edge0(comb0, ert0, src0, dst0)
    h, comb1, er1 = _mm1(np0, W1, alm, arm, exp8)
    np1 = _edge1(comb1, er1, src1, dst1)
    out = _fin(np1)
    return (out, h)
